# baseline (speedup 1.0000x reference)
import jax
import jax.numpy as jnp
from jax import lax
from jax.experimental import pallas as pl
from jax.experimental.pallas import tpu as pltpu
from jax.experimental.pallas import tpu_sc as plsc

N = 10000
E = 320000
D = 128

NC, NS = 2, 16
NW = NC * NS
CHUNK = 128
CPW = 80
E_PAD = NW * CPW * CHUNK
NPAD = 10240
ROWS_PER_TILE = NPAD // NS

BLK = 2000
GRID = N // BLK


_MESH = plsc.VectorSubcoreMesh(core_axis_name="c", subcore_axis_name="s")


DH = D // 2


NBUF = 5
AHEAD = 2

SLOW_CORE = 1
C_SLOW = 0
C_FAST = 2 * CPW - C_SLOW
C0 = C_SLOW if SLOW_CORE == 0 else C_FAST
C1 = C_SLOW if SLOW_CORE == 1 else C_FAST


def _ring_pipeline(table, src_v, dst_v, rows, gsems, ssems, acc_sh, cpw_c):
    for b in range(AHEAD):
        pltpu.async_copy(table.at[src_v.at[b]], rows.at[b], gsems.at[b])

    def group(g, carry):
        j0 = g * NBUF
        for b in range(NBUF):
            j = j0 + b
            ja = j + AHEAD
            ba = (b + AHEAD) % NBUF
            pltpu.make_async_copy(table.at[src_v.at[j]], rows.at[b],
                                  gsems.at[b]).wait()
            pltpu.async_copy(rows.at[b], acc_sh.at[dst_v.at[j]],
                             ssems.at[b], add=True)

            @pl.when(ja < cpw_c)
            def _(ja=ja, ba=ba):
                @pl.when(ja >= NBUF)
                def _():
                    pltpu.make_async_copy(
                        rows.at[ba], acc_sh.at[dst_v.at[ja]],
                        ssems.at[ba]).wait()
                pltpu.async_copy(table.at[src_v.at[ja]], rows.at[ba],
                                 gsems.at[ba])
        return carry

    lax.fori_loop(0, cpw_c // NBUF, group, 0)
    for b in range(NBUF):
        pltpu.make_async_copy(rows.at[b], acc_sh.at[dst_v.at[0]],
                              ssems.at[b]).wait()


def _sc_aggregate_body(table_a, table_b, src2d, dst2d, zeros_h,
                       out, src_v, dst_v, rows, gsems, ssems, acc_sh):
    cid = lax.axis_index("c")
    sid = lax.axis_index("s")
    zbase = sid * ROWS_PER_TILE

    if C0:
        @pl.when(cid == 0)
        def _():
            pltpu.sync_copy(src2d.at[pl.ds(sid * C0, C0)],
                            src_v.at[pl.ds(0, C0)])
            pltpu.sync_copy(dst2d.at[pl.ds(sid * C0, C0)],
                            dst_v.at[pl.ds(0, C0)])

    if C1:
        @pl.when(cid == 1)
        def _():
            base = NS * C0 + sid * C1
            pltpu.sync_copy(src2d.at[pl.ds(base, C1)],
                            src_v.at[pl.ds(0, C1)])
            pltpu.sync_copy(dst2d.at[pl.ds(base, C1)],
                            dst_v.at[pl.ds(0, C1)])

    pltpu.sync_copy(zeros_h.at[pl.ds(zbase, ROWS_PER_TILE)],
                    acc_sh.at[pl.ds(zbase, ROWS_PER_TILE)])
    plsc.subcore_barrier()

    for half, table in enumerate((table_a, table_b)):
        if C0:
            @pl.when(cid == 0)
            def _(table=table):
                _ring_pipeline(table, src_v, dst_v, rows, gsems, ssems,
                               acc_sh, C0)

        if C1:
            @pl.when(cid == 1)
            def _(table=table):
                _ring_pipeline(table, src_v, dst_v, rows, gsems, ssems,
                               acc_sh, C1)

        plsc.subcore_barrier()

        pltpu.sync_copy(acc_sh.at[pl.ds(zbase, ROWS_PER_TILE)],
                        out.at[cid, half, pl.ds(zbase, ROWS_PER_TILE)])
        if half == 0:
            pltpu.sync_copy(zeros_h.at[pl.ds(zbase, ROWS_PER_TILE)],
                            acc_sh.at[pl.ds(zbase, ROWS_PER_TILE)])
            plsc.subcore_barrier()


_sc_aggregate = pl.kernel(
    _sc_aggregate_body,
    jax.ShapeDtypeStruct((NC, 2, NPAD, DH), jnp.float32),
    mesh=_MESH,
    scratch_types=[
        pltpu.VMEM((C_FAST, CHUNK), jnp.int32),
        pltpu.VMEM((C_FAST, CHUNK), jnp.int32),
        pltpu.VMEM((NBUF, CHUNK, DH), jnp.float32),
        pltpu.SemaphoreType.DMA((NBUF,)),
        pltpu.SemaphoreType.DMA((NBUF,)),
        pltpu.VMEM_SHARED((NPAD, DH), jnp.float32),
    ],
    compiler_params=pltpu.CompilerParams(use_tc_tiling_on_sc=False),
)


def _sc_degree_body(dst2d, zeros_8, ones_8,
                    deg_out, dst_v, ones_v, deg_sh):
    cid = lax.axis_index("c")
    sid = lax.axis_index("s")
    wid = sid * NC + cid

    zbase = sid * ROWS_PER_TILE
    pltpu.sync_copy(zeros_8.at[pl.ds(zbase, ROWS_PER_TILE)],
                    deg_sh.at[pl.ds(zbase, ROWS_PER_TILE)])
    pltpu.sync_copy(dst2d.at[pl.ds(wid * CPW, CPW)], dst_v)
    pltpu.sync_copy(ones_8, ones_v)
    plsc.subcore_barrier()

    def step(j, carry):
        pltpu.sync_copy(ones_v, deg_sh.at[dst_v.at[j]], add=True)
        return carry

    lax.fori_loop(0, CPW, step, 0)
    plsc.subcore_barrier()

    pltpu.sync_copy(deg_sh.at[pl.ds(zbase, ROWS_PER_TILE)],
                    deg_out.at[cid, pl.ds(zbase, ROWS_PER_TILE)])


_sc_degree = pl.kernel(
    _sc_degree_body,
    jax.ShapeDtypeStruct((NC, NPAD, 16), jnp.float32),
    mesh=_MESH,
    scratch_types=[
        pltpu.VMEM((CPW, CHUNK), jnp.int32),
        pltpu.VMEM((CHUNK, 16), jnp.float32),
        pltpu.VMEM_SHARED((NPAD, 16), jnp.float32),
    ],
    compiler_params=pltpu.CompilerParams(use_tc_tiling_on_sc=False),
)


def _store_tl(h, wl_ref, tla_ref, tlb_ref):
    tl = jnp.dot(h, wl_ref[...], preferred_element_type=jnp.float32)
    tla_ref[...] = tl[:, :DH]
    tlb_ref[...] = tl[:, DH:]


def _tc_prep_body(x_ref, wl_ref, wr_ref, b_ref, tla_ref, tlb_ref, tr_ref):
    h = x_ref[...]
    _store_tl(h, wl_ref, tla_ref, tlb_ref)
    tr_ref[...] = (jnp.dot(h, wr_ref[...], preferred_element_type=jnp.float32)
                   + b_ref[...])


_W_SPEC = pl.BlockSpec((D, D), lambda i: (0, 0))
_B_SPEC = pl.BlockSpec((1, D), lambda i: (0, 0))
_H_SPEC = pl.BlockSpec((BLK, D), lambda i: (i, 0))
_HH_SPEC = pl.BlockSpec((BLK, DH), lambda i: (i, 0))
_AGG_SPEC = pl.BlockSpec((NC, 2, BLK, DH), lambda i: (0, 0, i, 0))
_DEG_SPEC = pl.BlockSpec((NC, BLK, 16), lambda i: (0, i, 0))

_TL_SHAPE = [jax.ShapeDtypeStruct((N, DH), jnp.float32)] * 2

_tc_prep = pl.pallas_call(
    _tc_prep_body,
    grid=(GRID,),
    in_specs=[_H_SPEC, _W_SPEC, _W_SPEC, _B_SPEC],
    out_specs=[_HH_SPEC, _HH_SPEC, _H_SPEC],
    out_shape=_TL_SHAPE + [jax.ShapeDtypeStruct((N, D), jnp.float32)],
)


def _mean_plus_tr(agg_ref, deg_ref, tr_ref):
    dsum = deg_ref[0] + deg_ref[1]
    inv = 1.0 / jnp.maximum(dsum[:, 0:1], 1.0)
    aggr = jnp.concatenate(
        [agg_ref[0, 0] + agg_ref[1, 0], agg_ref[0, 1] + agg_ref[1, 1]],
        axis=1)
    return aggr * inv + tr_ref[...]


def _tc_combine_body(agg_ref, deg_ref, tr_ref, wl_ref, wr_ref, b_ref,
                     tla_ref, tlb_ref, tr_o):
    h = jnp.maximum(_mean_plus_tr(agg_ref, deg_ref, tr_ref), 0.0)
    _store_tl(h, wl_ref, tla_ref, tlb_ref)
    tr_o[...] = (jnp.dot(h, wr_ref[...], preferred_element_type=jnp.float32)
                 + b_ref[...])


_tc_combine = pl.pallas_call(
    _tc_combine_body,
    grid=(GRID,),
    in_specs=[_AGG_SPEC, _DEG_SPEC, _H_SPEC, _W_SPEC, _W_SPEC, _B_SPEC],
    out_specs=[_HH_SPEC, _HH_SPEC, _H_SPEC],
    out_shape=_TL_SHAPE + [jax.ShapeDtypeStruct((N, D), jnp.float32)],
)


def _tc_final_body(agg_ref, deg_ref, tr_ref, out_ref):
    o = _mean_plus_tr(agg_ref, deg_ref, tr_ref)
    m = jnp.max(o, axis=1, keepdims=True)
    e = jnp.exp(o - m)
    lse = jnp.log(jnp.sum(e, axis=1, keepdims=True)) + m
    out_ref[...] = o - lse


_tc_final = pl.pallas_call(
    _tc_final_body,
    grid=(GRID,),
    in_specs=[_AGG_SPEC, _DEG_SPEC, _H_SPEC],
    out_specs=_H_SPEC,
    out_shape=jax.ShapeDtypeStruct((N, D), jnp.float32),
)


def kernel(x, edge_index, W1l, W1r, b1, W2l, W2r, b2, W3l, W3r, b3):
    src = edge_index[0].astype(jnp.int32)
    dst = edge_index[1].astype(jnp.int32)
    pad = E_PAD - E
    src2d = jnp.concatenate([src, jnp.zeros((pad,), jnp.int32)]
                            ).reshape(NW * CPW, CHUNK)
    dst2d = jnp.concatenate([dst, jnp.full((pad,), N, jnp.int32)]
                            ).reshape(NW * CPW, CHUNK)

    zeros_d = jnp.zeros((NPAD, DH), jnp.float32)
    zeros_16 = jnp.zeros((NPAD, 16), jnp.float32)
    ones_16 = jnp.ones((CHUNK, 16), jnp.float32)

    b1r = b1.reshape(1, D)
    b2r = b2.reshape(1, D)
    b3r = b3.reshape(1, D)

    deg = _sc_degree(dst2d, zeros_16, ones_16)
    t1la, t1lb, t1r = _tc_prep(x, W1l, W1r, b1r)
    agg1 = _sc_aggregate(t1la, t1lb, src2d, dst2d, zeros_d)
    t2la, t2lb, t2r = _tc_combine(agg1, deg, t1r, W2l, W2r, b2r)
    agg2 = _sc_aggregate(t2la, t2lb, src2d, dst2d, zeros_d)
    t3la, t3lb, t3r = _tc_combine(agg2, deg, t2r, W3l, W3r, b3r)
    agg3 = _sc_aggregate(t3la, t3lb, src2d, dst2d, zeros_d)
    return _tc_final(agg3, deg, t3r)


# speedup vs baseline: 1.0129x; 1.0129x over previous
import jax
import jax.numpy as jnp
from jax import lax
from jax.experimental import pallas as pl
from jax.experimental.pallas import tpu as pltpu
from jax.experimental.pallas import tpu_sc as plsc

N = 10000
E = 320000
D = 128

NC, NS = 2, 16
NW = NC * NS
CHUNK = 128
NPAD = 10240
ROWS_PER_TILE = NPAD // NS

BLK = 2000
GRID = N // BLK


_MESH = plsc.VectorSubcoreMesh(core_axis_name="c", subcore_axis_name="s")


DH = D // 2


NBUF = 6
AHEAD = 5

C0 = 132
C1 = 30
C_FAST = max(C0, C1)
CH_TOTAL = NS * (C0 + C1)
E_PAD = CH_TOTAL * CHUNK
CPW_DEG = CH_TOTAL // NW


def _ring_pipeline(table, src_v, dst_v, rows, gsems, ssems, acc_sh, cpw_c):
    for b in range(AHEAD):
        pltpu.async_copy(table.at[src_v.at[b]], rows.at[b], gsems.at[b])

    def group(g, carry):
        j0 = g * NBUF
        for b in range(NBUF):
            j = j0 + b
            ja = j + AHEAD
            ba = (b + AHEAD) % NBUF
            pltpu.make_async_copy(table.at[src_v.at[j]], rows.at[b],
                                  gsems.at[b]).wait()
            pltpu.async_copy(rows.at[b], acc_sh.at[dst_v.at[j]],
                             ssems.at[b], add=True)

            @pl.when(ja < cpw_c)
            def _(ja=ja, ba=ba):
                @pl.when(ja >= NBUF)
                def _():
                    pltpu.make_async_copy(
                        rows.at[ba], acc_sh.at[dst_v.at[ja]],
                        ssems.at[ba]).wait()
                pltpu.async_copy(table.at[src_v.at[ja]], rows.at[ba],
                                 gsems.at[ba])
        return carry

    lax.fori_loop(0, cpw_c // NBUF, group, 0)
    for b in range(NBUF):
        pltpu.make_async_copy(rows.at[b], acc_sh.at[dst_v.at[0]],
                              ssems.at[b]).wait()


def _sc_aggregate_body(table_a, table_b, src2d, dst2d, zeros_h,
                       out, src_v, dst_v, rows, gsems, ssems, acc_sh):
    cid = lax.axis_index("c")
    sid = lax.axis_index("s")
    zbase = sid * ROWS_PER_TILE

    if C0:
        @pl.when(cid == 0)
        def _():
            pltpu.sync_copy(src2d.at[pl.ds(sid * C0, C0)],
                            src_v.at[pl.ds(0, C0)])
            pltpu.sync_copy(dst2d.at[pl.ds(sid * C0, C0)],
                            dst_v.at[pl.ds(0, C0)])

    if C1:
        @pl.when(cid == 1)
        def _():
            base = NS * C0 + sid * C1
            pltpu.sync_copy(src2d.at[pl.ds(base, C1)],
                            src_v.at[pl.ds(0, C1)])
            pltpu.sync_copy(dst2d.at[pl.ds(base, C1)],
                            dst_v.at[pl.ds(0, C1)])

    pltpu.sync_copy(zeros_h.at[pl.ds(zbase, ROWS_PER_TILE)],
                    acc_sh.at[pl.ds(zbase, ROWS_PER_TILE)])
    plsc.subcore_barrier()

    for half, table in enumerate((table_a, table_b)):
        if C0:
            @pl.when(cid == 0)
            def _(table=table):
                _ring_pipeline(table, src_v, dst_v, rows, gsems, ssems,
                               acc_sh, C0)

        if C1:
            @pl.when(cid == 1)
            def _(table=table):
                _ring_pipeline(table, src_v, dst_v, rows, gsems, ssems,
                               acc_sh, C1)

        plsc.subcore_barrier()

        pltpu.sync_copy(acc_sh.at[pl.ds(zbase, ROWS_PER_TILE)],
                        out.at[cid, half, pl.ds(zbase, ROWS_PER_TILE)])
        if half == 0:
            pltpu.sync_copy(zeros_h.at[pl.ds(zbase, ROWS_PER_TILE)],
                            acc_sh.at[pl.ds(zbase, ROWS_PER_TILE)])
            plsc.subcore_barrier()


_sc_aggregate = pl.kernel(
    _sc_aggregate_body,
    jax.ShapeDtypeStruct((NC, 2, NPAD, DH), jnp.float32),
    mesh=_MESH,
    scratch_types=[
        pltpu.VMEM((C_FAST, CHUNK), jnp.int32),
        pltpu.VMEM((C_FAST, CHUNK), jnp.int32),
        pltpu.VMEM((NBUF, CHUNK, DH), jnp.float32),
        pltpu.SemaphoreType.DMA((NBUF,)),
        pltpu.SemaphoreType.DMA((NBUF,)),
        pltpu.VMEM_SHARED((NPAD, DH), jnp.float32),
    ],
    compiler_params=pltpu.CompilerParams(use_tc_tiling_on_sc=False),
)


def _sc_degree_body(dst2d, zeros_8, ones_8,
                    deg_out, dst_v, ones_v, deg_sh):
    cid = lax.axis_index("c")
    sid = lax.axis_index("s")
    wid = sid * NC + cid

    zbase = sid * ROWS_PER_TILE
    pltpu.sync_copy(zeros_8.at[pl.ds(zbase, ROWS_PER_TILE)],
                    deg_sh.at[pl.ds(zbase, ROWS_PER_TILE)])
    pltpu.sync_copy(dst2d.at[pl.ds(wid * CPW_DEG, CPW_DEG)], dst_v)
    pltpu.sync_copy(ones_8, ones_v)
    plsc.subcore_barrier()

    def step(j, carry):
        pltpu.sync_copy(ones_v, deg_sh.at[dst_v.at[j]], add=True)
        return carry

    lax.fori_loop(0, CPW_DEG, step, 0)
    plsc.subcore_barrier()

    pltpu.sync_copy(deg_sh.at[pl.ds(zbase, ROWS_PER_TILE)],
                    deg_out.at[cid, pl.ds(zbase, ROWS_PER_TILE)])


_sc_degree = pl.kernel(
    _sc_degree_body,
    jax.ShapeDtypeStruct((NC, NPAD, 16), jnp.float32),
    mesh=_MESH,
    scratch_types=[
        pltpu.VMEM((CPW_DEG, CHUNK), jnp.int32),
        pltpu.VMEM((CHUNK, 16), jnp.float32),
        pltpu.VMEM_SHARED((NPAD, 16), jnp.float32),
    ],
    compiler_params=pltpu.CompilerParams(use_tc_tiling_on_sc=False),
)


def _store_tl(h, wl_ref, tla_ref, tlb_ref):
    tl = jnp.dot(h, wl_ref[...], preferred_element_type=jnp.float32)
    tla_ref[...] = tl[:, :DH]
    tlb_ref[...] = tl[:, DH:]


def _tc_prep_body(x_ref, wl_ref, wr_ref, b_ref, tla_ref, tlb_ref, tr_ref):
    h = x_ref[...]
    _store_tl(h, wl_ref, tla_ref, tlb_ref)
    tr_ref[...] = (jnp.dot(h, wr_ref[...], preferred_element_type=jnp.float32)
                   + b_ref[...])


_W_SPEC = pl.BlockSpec((D, D), lambda i: (0, 0))
_B_SPEC = pl.BlockSpec((1, D), lambda i: (0, 0))
_H_SPEC = pl.BlockSpec((BLK, D), lambda i: (i, 0))
_HH_SPEC = pl.BlockSpec((BLK, DH), lambda i: (i, 0))
_AGG_SPEC = pl.BlockSpec((NC, 2, BLK, DH), lambda i: (0, 0, i, 0))
_DEG_SPEC = pl.BlockSpec((NC, BLK, 16), lambda i: (0, i, 0))

_TL_SHAPE = [jax.ShapeDtypeStruct((N, DH), jnp.float32)] * 2

_tc_prep = pl.pallas_call(
    _tc_prep_body,
    grid=(GRID,),
    in_specs=[_H_SPEC, _W_SPEC, _W_SPEC, _B_SPEC],
    out_specs=[_HH_SPEC, _HH_SPEC, _H_SPEC],
    out_shape=_TL_SHAPE + [jax.ShapeDtypeStruct((N, D), jnp.float32)],
)


def _mean_plus_tr(agg_ref, deg_ref, tr_ref):
    dsum = deg_ref[0] + deg_ref[1]
    inv = 1.0 / jnp.maximum(dsum[:, 0:1], 1.0)
    aggr = jnp.concatenate(
        [agg_ref[0, 0] + agg_ref[1, 0], agg_ref[0, 1] + agg_ref[1, 1]],
        axis=1)
    return aggr * inv + tr_ref[...]


def _tc_combine_body(agg_ref, deg_ref, tr_ref, wl_ref, wr_ref, b_ref,
                     tla_ref, tlb_ref, tr_o):
    h = jnp.maximum(_mean_plus_tr(agg_ref, deg_ref, tr_ref), 0.0)
    _store_tl(h, wl_ref, tla_ref, tlb_ref)
    tr_o[...] = (jnp.dot(h, wr_ref[...], preferred_element_type=jnp.float32)
                 + b_ref[...])


_tc_combine = pl.pallas_call(
    _tc_combine_body,
    grid=(GRID,),
    in_specs=[_AGG_SPEC, _DEG_SPEC, _H_SPEC, _W_SPEC, _W_SPEC, _B_SPEC],
    out_specs=[_HH_SPEC, _HH_SPEC, _H_SPEC],
    out_shape=_TL_SHAPE + [jax.ShapeDtypeStruct((N, D), jnp.float32)],
)


def _tc_final_body(agg_ref, deg_ref, tr_ref, out_ref):
    o = _mean_plus_tr(agg_ref, deg_ref, tr_ref)
    m = jnp.max(o, axis=1, keepdims=True)
    e = jnp.exp(o - m)
    lse = jnp.log(jnp.sum(e, axis=1, keepdims=True)) + m
    out_ref[...] = o - lse


_tc_final = pl.pallas_call(
    _tc_final_body,
    grid=(GRID,),
    in_specs=[_AGG_SPEC, _DEG_SPEC, _H_SPEC],
    out_specs=_H_SPEC,
    out_shape=jax.ShapeDtypeStruct((N, D), jnp.float32),
)


def kernel(x, edge_index, W1l, W1r, b1, W2l, W2r, b2, W3l, W3r, b3):
    src = edge_index[0].astype(jnp.int32)
    dst = edge_index[1].astype(jnp.int32)
    pad = E_PAD - E
    src2d = jnp.concatenate([src, jnp.zeros((pad,), jnp.int32)]
                            ).reshape(CH_TOTAL, CHUNK)
    dst2d = jnp.concatenate([dst, jnp.full((pad,), N, jnp.int32)]
                            ).reshape(CH_TOTAL, CHUNK)

    zeros_d = jnp.zeros((NPAD, DH), jnp.float32)
    zeros_16 = jnp.zeros((NPAD, 16), jnp.float32)
    ones_16 = jnp.ones((CHUNK, 16), jnp.float32)

    b1r = b1.reshape(1, D)
    b2r = b2.reshape(1, D)
    b3r = b3.reshape(1, D)

    deg = _sc_degree(dst2d, zeros_16, ones_16)
    t1la, t1lb, t1r = _tc_prep(x, W1l, W1r, b1r)
    agg1 = _sc_aggregate(t1la, t1lb, src2d, dst2d, zeros_d)
    t2la, t2lb, t2r = _tc_combine(agg1, deg, t1r, W2l, W2r, b2r)
    agg2 = _sc_aggregate(t2la, t2lb, src2d, dst2d, zeros_d)
    t3la, t3lb, t3r = _tc_combine(agg2, deg, t2r, W3l, W3r, b3r)
    agg3 = _sc_aggregate(t3la, t3lb, src2d, dst2d, zeros_d)
    return _tc_final(agg3, deg, t3r)


# speedup vs baseline: 1.2451x; 1.2292x over previous
import jax
import jax.numpy as jnp
from jax import lax
from jax.experimental import pallas as pl
from jax.experimental.pallas import tpu as pltpu
from jax.experimental.pallas import tpu_sc as plsc

N = 10000
E = 320000
D = 128

NC, NS = 2, 16
NW = NC * NS
CHUNK = 128
NPAD = 10240
ROWS_PER_TILE = NPAD // NS

BLK = 2000
GRID = N // BLK


_MESH = plsc.VectorSubcoreMesh(core_axis_name="c", subcore_axis_name="s")


DH = D // 2


NBUF = 5
AHEAD = 4

C0 = 130
C1 = 30
C_FAST = max(C0, C1)
CH_TOTAL = NS * (C0 + C1)
E_PAD = CH_TOTAL * CHUNK
CPW_DEG = CH_TOTAL // NW


def _ring_pipeline(table, src_v, dst_v, rows, gsems, ssems, acc_sh, cpw_c):
    for b in range(AHEAD):
        pltpu.async_copy(table.at[src_v.at[b]], rows.at[b], gsems.at[b])

    def group(g, carry):
        j0 = g * NBUF
        for b in range(NBUF):
            j = j0 + b
            ja = j + AHEAD
            ba = (b + AHEAD) % NBUF
            pltpu.make_async_copy(table.at[src_v.at[j]], rows.at[b],
                                  gsems.at[b]).wait()
            pltpu.async_copy(rows.at[b], acc_sh.at[dst_v.at[j]],
                             ssems.at[b], add=True)

            @pl.when(ja < cpw_c)
            def _(ja=ja, ba=ba):
                @pl.when(ja >= NBUF)
                def _():
                    pltpu.make_async_copy(
                        rows.at[ba], acc_sh.at[dst_v.at[ja]],
                        ssems.at[ba]).wait()
                pltpu.async_copy(table.at[src_v.at[ja]], rows.at[ba],
                                 gsems.at[ba])
        return carry

    lax.fori_loop(0, cpw_c // NBUF, group, 0)
    for b in range(NBUF):
        pltpu.make_async_copy(rows.at[b], acc_sh.at[dst_v.at[0]],
                              ssems.at[b]).wait()


def _sc_aggregate_body(table_a, table_b, src2d, dst2d, zeros_h,
                       out, src_v, dst_v, rows, gsems, ssems, acc_sh):
    cid = lax.axis_index("c")
    sid = lax.axis_index("s")
    zbase = sid * ROWS_PER_TILE

    if C0:
        @pl.when(cid == 0)
        def _():
            pltpu.sync_copy(src2d.at[pl.ds(sid * C0, C0)],
                            src_v.at[pl.ds(0, C0)])
            pltpu.sync_copy(dst2d.at[pl.ds(sid * C0, C0)],
                            dst_v.at[pl.ds(0, C0)])

    if C1:
        @pl.when(cid == 1)
        def _():
            base = NS * C0 + sid * C1
            pltpu.sync_copy(src2d.at[pl.ds(base, C1)],
                            src_v.at[pl.ds(0, C1)])
            pltpu.sync_copy(dst2d.at[pl.ds(base, C1)],
                            dst_v.at[pl.ds(0, C1)])

    pltpu.sync_copy(zeros_h.at[pl.ds(zbase, ROWS_PER_TILE)],
                    acc_sh.at[pl.ds(zbase, ROWS_PER_TILE)])
    plsc.subcore_barrier()

    for half, table in enumerate((table_a, table_b)):
        if C0:
            @pl.when(cid == 0)
            def _(table=table):
                _ring_pipeline(table, src_v, dst_v, rows, gsems, ssems,
                               acc_sh, C0)

        if C1:
            @pl.when(cid == 1)
            def _(table=table):
                _ring_pipeline(table, src_v, dst_v, rows, gsems, ssems,
                               acc_sh, C1)

        plsc.subcore_barrier()

        pltpu.sync_copy(acc_sh.at[pl.ds(zbase, ROWS_PER_TILE)],
                        out.at[cid, half, pl.ds(zbase, ROWS_PER_TILE)])
        if half == 0:
            pltpu.sync_copy(zeros_h.at[pl.ds(zbase, ROWS_PER_TILE)],
                            acc_sh.at[pl.ds(zbase, ROWS_PER_TILE)])
            plsc.subcore_barrier()


_sc_aggregate = pl.kernel(
    _sc_aggregate_body,
    jax.ShapeDtypeStruct((NC, 2, NPAD, DH), jnp.float32),
    mesh=_MESH,
    scratch_types=[
        pltpu.VMEM((C_FAST, CHUNK), jnp.int32),
        pltpu.VMEM((C_FAST, CHUNK), jnp.int32),
        pltpu.VMEM((NBUF, CHUNK, DH), jnp.float32),
        pltpu.SemaphoreType.DMA((NBUF,)),
        pltpu.SemaphoreType.DMA((NBUF,)),
        pltpu.VMEM_SHARED((NPAD, DH), jnp.float32),
    ],
    compiler_params=pltpu.CompilerParams(use_tc_tiling_on_sc=False),
)


def _sc_degree_body(dst2d, zeros_8, ones_8,
                    deg_out, dst_v, ones_v, deg_sh):
    cid = lax.axis_index("c")
    sid = lax.axis_index("s")
    wid = sid * NC + cid

    zbase = sid * ROWS_PER_TILE
    pltpu.sync_copy(zeros_8.at[pl.ds(zbase, ROWS_PER_TILE)],
                    deg_sh.at[pl.ds(zbase, ROWS_PER_TILE)])
    pltpu.sync_copy(dst2d.at[pl.ds(wid * CPW_DEG, CPW_DEG)], dst_v)
    pltpu.sync_copy(ones_8, ones_v)
    plsc.subcore_barrier()

    def step(j, carry):
        pltpu.sync_copy(ones_v, deg_sh.at[dst_v.at[j]], add=True)
        return carry

    lax.fori_loop(0, CPW_DEG, step, 0)
    plsc.subcore_barrier()

    pltpu.sync_copy(deg_sh.at[pl.ds(zbase, ROWS_PER_TILE)],
                    deg_out.at[cid, pl.ds(zbase, ROWS_PER_TILE)])


_sc_degree = pl.kernel(
    _sc_degree_body,
    jax.ShapeDtypeStruct((NC, NPAD, 16), jnp.float32),
    mesh=_MESH,
    scratch_types=[
        pltpu.VMEM((CPW_DEG, CHUNK), jnp.int32),
        pltpu.VMEM((CHUNK, 16), jnp.float32),
        pltpu.VMEM_SHARED((NPAD, 16), jnp.float32),
    ],
    compiler_params=pltpu.CompilerParams(use_tc_tiling_on_sc=False),
)


def _store_tl(h, wl_ref, tla_ref, tlb_ref):
    tl = jnp.dot(h, wl_ref[...], preferred_element_type=jnp.float32)
    tla_ref[...] = tl[:, :DH]
    tlb_ref[...] = tl[:, DH:]


def _tc_prep_body(x_ref, wl_ref, wr_ref, b_ref, tla_ref, tlb_ref, tr_ref):
    h = x_ref[...]
    _store_tl(h, wl_ref, tla_ref, tlb_ref)
    tr_ref[...] = (jnp.dot(h, wr_ref[...], preferred_element_type=jnp.float32)
                   + b_ref[...])


_W_SPEC = pl.BlockSpec((D, D), lambda i: (0, 0))
_B_SPEC = pl.BlockSpec((1, D), lambda i: (0, 0))
_H_SPEC = pl.BlockSpec((BLK, D), lambda i: (i, 0))
_HH_SPEC = pl.BlockSpec((BLK, DH), lambda i: (i, 0))
_AGG_SPEC = pl.BlockSpec((NC, 2, BLK, DH), lambda i: (0, 0, i, 0))
_DEG_SPEC = pl.BlockSpec((NC, BLK, 16), lambda i: (0, i, 0))

_TL_SHAPE = [jax.ShapeDtypeStruct((N, DH), jnp.float32)] * 2

_tc_prep = pl.pallas_call(
    _tc_prep_body,
    grid=(GRID,),
    in_specs=[_H_SPEC, _W_SPEC, _W_SPEC, _B_SPEC],
    out_specs=[_HH_SPEC, _HH_SPEC, _H_SPEC],
    out_shape=_TL_SHAPE + [jax.ShapeDtypeStruct((N, D), jnp.float32)],
)


def _mean_plus_tr(agg_ref, deg_ref, tr_ref):
    dsum = deg_ref[0] + deg_ref[1]
    inv = 1.0 / jnp.maximum(dsum[:, 0:1], 1.0)
    aggr = jnp.concatenate(
        [agg_ref[0, 0] + agg_ref[1, 0], agg_ref[0, 1] + agg_ref[1, 1]],
        axis=1)
    return aggr * inv + tr_ref[...]


def _tc_combine_body(agg_ref, deg_ref, tr_ref, wl_ref, wr_ref, b_ref,
                     tla_ref, tlb_ref, tr_o):
    h = jnp.maximum(_mean_plus_tr(agg_ref, deg_ref, tr_ref), 0.0)
    _store_tl(h, wl_ref, tla_ref, tlb_ref)
    tr_o[...] = (jnp.dot(h, wr_ref[...], preferred_element_type=jnp.float32)
                 + b_ref[...])


_tc_combine = pl.pallas_call(
    _tc_combine_body,
    grid=(GRID,),
    in_specs=[_AGG_SPEC, _DEG_SPEC, _H_SPEC, _W_SPEC, _W_SPEC, _B_SPEC],
    out_specs=[_HH_SPEC, _HH_SPEC, _H_SPEC],
    out_shape=_TL_SHAPE + [jax.ShapeDtypeStruct((N, D), jnp.float32)],
)


def _tc_final_body(agg_ref, deg_ref, tr_ref, out_ref):
    o = _mean_plus_tr(agg_ref, deg_ref, tr_ref)
    m = jnp.max(o, axis=1, keepdims=True)
    e = jnp.exp(o - m)
    lse = jnp.log(jnp.sum(e, axis=1, keepdims=True)) + m
    out_ref[...] = o - lse


_tc_final = pl.pallas_call(
    _tc_final_body,
    grid=(GRID,),
    in_specs=[_AGG_SPEC, _DEG_SPEC, _H_SPEC],
    out_specs=_H_SPEC,
    out_shape=jax.ShapeDtypeStruct((N, D), jnp.float32),
)


def kernel(x, edge_index, W1l, W1r, b1, W2l, W2r, b2, W3l, W3r, b3):
    src = edge_index[0].astype(jnp.int32)
    dst = edge_index[1].astype(jnp.int32)
    pad = E_PAD - E
    src2d = jnp.concatenate([src, jnp.zeros((pad,), jnp.int32)]
                            ).reshape(CH_TOTAL, CHUNK)
    dst2d = jnp.concatenate([dst, jnp.full((pad,), N, jnp.int32)]
                            ).reshape(CH_TOTAL, CHUNK)

    zeros_d = jnp.zeros((NPAD, DH), jnp.float32)
    zeros_16 = jnp.zeros((NPAD, 16), jnp.float32)
    ones_16 = jnp.ones((CHUNK, 16), jnp.float32)

    b1r = b1.reshape(1, D)
    b2r = b2.reshape(1, D)
    b3r = b3.reshape(1, D)

    deg = _sc_degree(dst2d, zeros_16, ones_16)
    t1la, t1lb, t1r = _tc_prep(x, W1l, W1r, b1r)
    agg1 = _sc_aggregate(t1la, t1lb, src2d, dst2d, zeros_d)
    t2la, t2lb, t2r = _tc_combine(agg1, deg, t1r, W2l, W2r, b2r)
    agg2 = _sc_aggregate(t2la, t2lb, src2d, dst2d, zeros_d)
    t3la, t3lb, t3r = _tc_combine(agg2, deg, t2r, W3l, W3r, b3r)
    agg3 = _sc_aggregate(t3la, t3lb, src2d, dst2d, zeros_d)
    return _tc_final(agg3, deg, t3r)


# speedup vs baseline: 2.1733x; 1.7456x over previous
import jax
import jax.numpy as jnp
from jax import lax
from jax.experimental import pallas as pl
from jax.experimental.pallas import tpu as pltpu
from jax.experimental.pallas import tpu_sc as plsc

N = 10000
E = 320000
D = 128

NC, NS = 2, 16
NW = NC * NS
CHUNK = 128
NPAD = 10240
ROWS_PER_TILE = NPAD // NS

BLK = 2000
GRID = N // BLK


_MESH = plsc.VectorSubcoreMesh(core_axis_name="c", subcore_axis_name="s")


DH = D // 2


NBUF = 5
AHEAD = 2

C0 = 130
C1 = 30
C_FAST = max(C0, C1)
CH_TOTAL = NS * (C0 + C1)
E_PAD = CH_TOTAL * CHUNK
CPW_DEG = CH_TOTAL // NW


def _ring_pipeline(table, src_v, dst_v, rows, gsems, ssems, acc_sh, cpw_c):
    for b in range(AHEAD):
        pltpu.async_copy(table.at[src_v.at[b]], rows.at[b], gsems.at[b])

    def group(g, carry):
        j0 = g * NBUF
        for b in range(NBUF):
            j = j0 + b
            ja = j + AHEAD
            ba = (b + AHEAD) % NBUF
            pltpu.make_async_copy(table.at[src_v.at[j]], rows.at[b],
                                  gsems.at[b]).wait()
            pltpu.async_copy(rows.at[b], acc_sh.at[dst_v.at[j]],
                             ssems.at[b], add=True)

            @pl.when(ja < cpw_c)
            def _(ja=ja, ba=ba):
                @pl.when(ja >= NBUF)
                def _():
                    pltpu.make_async_copy(
                        rows.at[ba], acc_sh.at[dst_v.at[ja]],
                        ssems.at[ba]).wait()
                pltpu.async_copy(table.at[src_v.at[ja]], rows.at[ba],
                                 gsems.at[ba])
        return carry

    lax.fori_loop(0, cpw_c // NBUF, group, 0)
    for b in range(NBUF):
        pltpu.make_async_copy(rows.at[b], acc_sh.at[dst_v.at[0]],
                              ssems.at[b]).wait()


def _sc_aggregate_body(table, src2d, dst2d, zeros_h,
                       out, src_v, dst_v, rows, gsems, ssems, acc_sh):
    cid = lax.axis_index("c")
    sid = lax.axis_index("s")
    zbase = sid * ROWS_PER_TILE

    @pl.when(cid == 0)
    def _():
        pltpu.sync_copy(src2d.at[pl.ds(sid * C0, C0)],
                        src_v.at[pl.ds(0, C0)])
        pltpu.sync_copy(dst2d.at[pl.ds(sid * C0, C0)],
                        dst_v.at[pl.ds(0, C0)])

    @pl.when(cid == 1)
    def _():
        base = NS * C0 + sid * C1
        pltpu.sync_copy(src2d.at[pl.ds(base, C1)],
                        src_v.at[pl.ds(0, C1)])
        pltpu.sync_copy(dst2d.at[pl.ds(base, C1)],
                        dst_v.at[pl.ds(0, C1)])

    pltpu.sync_copy(zeros_h.at[pl.ds(zbase, ROWS_PER_TILE)],
                    acc_sh.at[pl.ds(zbase, ROWS_PER_TILE)])
    plsc.subcore_barrier()

    @pl.when(cid == 0)
    def _():
        _ring_pipeline(table, src_v, dst_v, rows, gsems, ssems, acc_sh, C0)

    @pl.when(cid == 1)
    def _():
        _ring_pipeline(table, src_v, dst_v, rows, gsems, ssems, acc_sh, C1)

    plsc.subcore_barrier()

    pltpu.sync_copy(acc_sh.at[pl.ds(zbase, ROWS_PER_TILE)],
                    out.at[cid, pl.ds(zbase, ROWS_PER_TILE)])


_sc_aggregate = pl.kernel(
    _sc_aggregate_body,
    jax.ShapeDtypeStruct((NC, NPAD, D), jnp.bfloat16),
    mesh=_MESH,
    scratch_types=[
        pltpu.VMEM((C_FAST, CHUNK), jnp.int32),
        pltpu.VMEM((C_FAST, CHUNK), jnp.int32),
        pltpu.VMEM((NBUF, CHUNK, D), jnp.bfloat16),
        pltpu.SemaphoreType.DMA((NBUF,)),
        pltpu.SemaphoreType.DMA((NBUF,)),
        pltpu.VMEM_SHARED((NPAD, D), jnp.bfloat16),
    ],
    compiler_params=pltpu.CompilerParams(use_tc_tiling_on_sc=False),
)


def _sc_degree_body(dst2d, zeros_8, ones_8,
                    deg_out, dst_v, ones_v, deg_sh):
    cid = lax.axis_index("c")
    sid = lax.axis_index("s")
    wid = sid * NC + cid

    zbase = sid * ROWS_PER_TILE
    pltpu.sync_copy(zeros_8.at[pl.ds(zbase, ROWS_PER_TILE)],
                    deg_sh.at[pl.ds(zbase, ROWS_PER_TILE)])
    pltpu.sync_copy(dst2d.at[pl.ds(wid * CPW_DEG, CPW_DEG)], dst_v)
    pltpu.sync_copy(ones_8, ones_v)
    plsc.subcore_barrier()

    def step(j, carry):
        pltpu.sync_copy(ones_v, deg_sh.at[dst_v.at[j]], add=True)
        return carry

    lax.fori_loop(0, CPW_DEG, step, 0)
    plsc.subcore_barrier()

    pltpu.sync_copy(deg_sh.at[pl.ds(zbase, ROWS_PER_TILE)],
                    deg_out.at[cid, pl.ds(zbase, ROWS_PER_TILE)])


_sc_degree = pl.kernel(
    _sc_degree_body,
    jax.ShapeDtypeStruct((NC, NPAD, 16), jnp.float32),
    mesh=_MESH,
    scratch_types=[
        pltpu.VMEM((CPW_DEG, CHUNK), jnp.int32),
        pltpu.VMEM((CHUNK, 16), jnp.float32),
        pltpu.VMEM_SHARED((NPAD, 16), jnp.float32),
    ],
    compiler_params=pltpu.CompilerParams(use_tc_tiling_on_sc=False),
)


def _tc_prep_body(x_ref, wl_ref, wr_ref, b_ref, tl_ref, tr_ref):
    h = x_ref[...]
    tl_ref[...] = jnp.dot(h, wl_ref[...],
                          preferred_element_type=jnp.float32
                          ).astype(jnp.bfloat16)
    tr_ref[...] = (jnp.dot(h, wr_ref[...], preferred_element_type=jnp.float32)
                   + b_ref[...])


_W_SPEC = pl.BlockSpec((D, D), lambda i: (0, 0))
_B_SPEC = pl.BlockSpec((1, D), lambda i: (0, 0))
_H_SPEC = pl.BlockSpec((BLK, D), lambda i: (i, 0))
_AGG_SPEC = pl.BlockSpec((NC, BLK, D), lambda i: (0, i, 0))
_DEG_SPEC = pl.BlockSpec((NC, BLK, 16), lambda i: (0, i, 0))

_TL_SHAPE = jax.ShapeDtypeStruct((N, D), jnp.bfloat16)

_tc_prep = pl.pallas_call(
    _tc_prep_body,
    grid=(GRID,),
    in_specs=[_H_SPEC, _W_SPEC, _W_SPEC, _B_SPEC],
    out_specs=[_H_SPEC, _H_SPEC],
    out_shape=[_TL_SHAPE, jax.ShapeDtypeStruct((N, D), jnp.float32)],
)


def _mean_plus_tr(agg_ref, deg_ref, tr_ref):
    dsum = deg_ref[0] + deg_ref[1]
    inv = 1.0 / jnp.maximum(dsum[:, 0:1], 1.0)
    aggr = (agg_ref[0].astype(jnp.float32)
            + agg_ref[1].astype(jnp.float32))
    return aggr * inv + tr_ref[...]


def _tc_combine_body(agg_ref, deg_ref, tr_ref, wl_ref, wr_ref, b_ref,
                     tl_ref, tr_o):
    h = jnp.maximum(_mean_plus_tr(agg_ref, deg_ref, tr_ref), 0.0)
    tl_ref[...] = jnp.dot(h, wl_ref[...],
                          preferred_element_type=jnp.float32
                          ).astype(jnp.bfloat16)
    tr_o[...] = (jnp.dot(h, wr_ref[...], preferred_element_type=jnp.float32)
                 + b_ref[...])


_tc_combine = pl.pallas_call(
    _tc_combine_body,
    grid=(GRID,),
    in_specs=[_AGG_SPEC, _DEG_SPEC, _H_SPEC, _W_SPEC, _W_SPEC, _B_SPEC],
    out_specs=[_H_SPEC, _H_SPEC],
    out_shape=[_TL_SHAPE, jax.ShapeDtypeStruct((N, D), jnp.float32)],
)


def _tc_final_body(agg_ref, deg_ref, tr_ref, out_ref):
    o = _mean_plus_tr(agg_ref, deg_ref, tr_ref)
    m = jnp.max(o, axis=1, keepdims=True)
    e = jnp.exp(o - m)
    lse = jnp.log(jnp.sum(e, axis=1, keepdims=True)) + m
    out_ref[...] = o - lse


_tc_final = pl.pallas_call(
    _tc_final_body,
    grid=(GRID,),
    in_specs=[_AGG_SPEC, _DEG_SPEC, _H_SPEC],
    out_specs=_H_SPEC,
    out_shape=jax.ShapeDtypeStruct((N, D), jnp.float32),
)


def kernel(x, edge_index, W1l, W1r, b1, W2l, W2r, b2, W3l, W3r, b3):
    src = edge_index[0].astype(jnp.int32)
    dst = edge_index[1].astype(jnp.int32)
    pad = E_PAD - E
    src2d = jnp.concatenate([src, jnp.zeros((pad,), jnp.int32)]
                            ).reshape(CH_TOTAL, CHUNK)
    dst2d = jnp.concatenate([dst, jnp.full((pad,), N, jnp.int32)]
                            ).reshape(CH_TOTAL, CHUNK)

    zeros_d = jnp.zeros((NPAD, D), jnp.bfloat16)
    zeros_16 = jnp.zeros((NPAD, 16), jnp.float32)
    ones_16 = jnp.ones((CHUNK, 16), jnp.float32)

    b1r = b1.reshape(1, D)
    b2r = b2.reshape(1, D)
    b3r = b3.reshape(1, D)

    deg = _sc_degree(dst2d, zeros_16, ones_16)
    t1l, t1r = _tc_prep(x, W1l, W1r, b1r)
    agg1 = _sc_aggregate(t1l, src2d, dst2d, zeros_d)
    t2l, t2r = _tc_combine(agg1, deg, t1r, W2l, W2r, b2r)
    agg2 = _sc_aggregate(t2l, src2d, dst2d, zeros_d)
    t3l, t3r = _tc_combine(agg2, deg, t2r, W3l, W3r, b3r)
    agg3 = _sc_aggregate(t3l, src2d, dst2d, zeros_d)
    return _tc_final(agg3, deg, t3r)
